# baseline (speedup 1.0000x reference)
import functools

import jax
import jax.numpy as jnp
from jax import lax
from jax.experimental import pallas as pl
from jax.experimental.pallas import tpu as pltpu
from jax.experimental.pallas import tpu_sc as plsc

V = 1000000
E = 16
B = 16384
S = 200

NC = 2
NS = 16
NW = NC * NS
ROWS_W = B // NW
CH = 128
NCHUNK = ROWS_W // CH
NPASS = 4
SH = S // NPASS
CSZH = CH * SH

VP = 1048576
STAGE = VP // NS
STAGE_SUB = STAGE // 8


TC_BLK = 262144


def _tc_body(tab_ref, w_ref, bias_ref, out_ref):
    out_ref[...] = (
        jnp.sum(tab_ref[...] * w_ref[...], axis=0) + bias_ref[0, 0]
    )


def _fold_table(table_t, w_col, bias):
    return pl.pallas_call(
        _tc_body,
        grid=(VP // TC_BLK,),
        in_specs=[
            pl.BlockSpec((E, TC_BLK), lambda k: (0, k)),
            pl.BlockSpec((E, 1), lambda k: (0, 0)),
            pl.BlockSpec(memory_space=pltpu.SMEM),
        ],
        out_specs=pl.BlockSpec((TC_BLK,), lambda k: (k,)),
        out_shape=jax.ShapeDtypeStruct((VP,), jnp.float32),
    )(table_t, w_col, bias)


_sc_mesh = plsc.VectorSubcoreMesh(core_axis_name="c", subcore_axis_name="s")


@functools.partial(
    pl.kernel,
    out_type=jax.ShapeDtypeStruct((B,), jnp.float32),
    mesh=_sc_mesh,
    scratch_types=[
        pltpu.VMEM((S, CH), jnp.int32),
        pltpu.VMEM((CSZH,), jnp.int32),
        pltpu.VMEM((CSZH,), jnp.int32),
        pltpu.VMEM((CSZH,), jnp.float32),
        pltpu.VMEM((CSZH,), jnp.float32),
        pltpu.VMEM((STAGE_SUB,), jnp.float32),
        pltpu.VMEM((STAGE_SUB,), jnp.float32),
        pltpu.VMEM((CH,), jnp.float32),
        pltpu.VMEM_SHARED((VP,), jnp.float32),
        pltpu.SemaphoreType.DMA,
        pltpu.SemaphoreType.DMA,
        pltpu.SemaphoreType.DMA,
    ],
    compiler_params=pltpu.CompilerParams(needs_layout_passes=False),
)
def _sc_pool(idx_hbm, t2_hbm, out_hbm, idx_t, idx_v0, idx_v1, vals_v0,
             vals_v1, stage_v0, stage_v1, out_v, t2_sh, sem, sem2, sem3):
    cid = lax.axis_index("c")
    sid = lax.axis_index("s")
    wid = sid * NC + cid
    idx_vs = (idx_v0, idx_v1)
    vals_vs = (vals_v0, vals_v1)

    def fire_idx(c):
        col0 = wid * ROWS_W + c * CH
        return pltpu.async_copy(idx_hbm.at[:, pl.ds(col0, CH)], idx_t, sem2)

    NG = CH // 16

    def do_repack(p, iv):
        def repack(s, carry):
            for g in range(NG):
                iv[pl.ds(s * CH + g * 16, 16)] = (
                    idx_t[p * SH + s, pl.ds(g * 16, 16)])
            return carry

        lax.fori_loop(0, SH, repack, 0)

    def reduce_pass(b, accs):
        vv = vals_vs[b]

        def body(s, a):
            return tuple(a[g] + vv[pl.ds(s * CH + g * 16, 16)]
                         for g in range(NG))

        return list(lax.fori_loop(0, SH, body, tuple(accs)))

    idx_dma = fire_idx(0)
    idx_dma.wait()
    do_repack(0, idx_vs[0])
    do_repack(1, idx_vs[1])

    for k in range(STAGE // STAGE_SUB):
        off = sid * STAGE + k * STAGE_SUB
        pltpu.sync_copy(t2_hbm.at[pl.ds(off, STAGE_SUB)], stage_v0)
        pltpu.sync_copy(stage_v0, t2_sh.at[pl.ds(off, STAGE_SUB)])

    plsc.subcore_barrier()

    for c in range(NCHUNK):
        col0 = wid * ROWS_W + c * CH
        accs = [jnp.zeros((16,), jnp.float32)] * NG

        gat = None
        for p in range(NPASS):
            b = p & 1
            if not (c == 0 and p < 2):
                do_repack(p, idx_vs[b])
            d = pltpu.async_copy(t2_sh.at[idx_vs[b]], vals_vs[b], sem)
            if p == NPASS - 1 and c < NCHUNK - 1:
                idx_dma = fire_idx(c + 1)
            if gat is not None:
                gat.wait()
                accs = reduce_pass(1 - b, accs)
            gat = d
        gat.wait()
        accs = reduce_pass((NPASS - 1) & 1, accs)

        for g in range(NG):
            out_v[pl.ds(g * 16, 16)] = 1.0 / (1.0 + jnp.exp(-accs[g]))
        pltpu.sync_copy(out_v, out_hbm.at[pl.ds(col0, CH)])
        if c < NCHUNK - 1:
            idx_dma.wait()


def kernel(inputs, table, dense_w, dense_b):
    w_col = dense_w * (1.0 / S)
    bias = (dense_b * (1.0 / S)).reshape(1, 1)

    t2 = _fold_table(table.T, w_col, bias)

    out = _sc_pool(inputs.T, t2)
    return out.reshape(B, 1)


# speedup vs baseline: 1.0198x; 1.0198x over previous
import functools

import jax
import jax.numpy as jnp
from jax import lax
from jax.experimental import pallas as pl
from jax.experimental.pallas import tpu as pltpu
from jax.experimental.pallas import tpu_sc as plsc

V = 1000000
E = 16
B = 16384
S = 200

NC = 2
NS = 16
NW = NC * NS
ROWS_W = B // NW
CH = 128
NCHUNK = ROWS_W // CH
NPASS = 4
SH = S // NPASS
CSZH = CH * SH

VP = 1048576
STAGE = VP // NS
STAGE_SUB = STAGE // 8


TC_BLK = 262144


def _tc_body(tab_ref, w_ref, bias_ref, out_ref):
    out_ref[...] = (
        jnp.sum(tab_ref[...] * w_ref[...], axis=0) + bias_ref[0, 0]
    )


def _fold_table(table_t, w_col, bias):
    return pl.pallas_call(
        _tc_body,
        grid=(VP // TC_BLK,),
        in_specs=[
            pl.BlockSpec((E, TC_BLK), lambda k: (0, k)),
            pl.BlockSpec((E, 1), lambda k: (0, 0)),
            pl.BlockSpec(memory_space=pltpu.SMEM),
        ],
        out_specs=pl.BlockSpec((TC_BLK,), lambda k: (k,)),
        out_shape=jax.ShapeDtypeStruct((VP,), jnp.float32),
    )(table_t, w_col, bias)


_sc_mesh = plsc.VectorSubcoreMesh(core_axis_name="c", subcore_axis_name="s")


@functools.partial(
    pl.kernel,
    out_type=jax.ShapeDtypeStruct((B,), jnp.float32),
    mesh=_sc_mesh,
    scratch_types=[
        pltpu.VMEM((S, CH), jnp.int32),
        pltpu.VMEM((CSZH,), jnp.int32),
        pltpu.VMEM((CSZH,), jnp.int32),
        pltpu.VMEM((CSZH,), jnp.float32),
        pltpu.VMEM((CSZH,), jnp.float32),
        pltpu.VMEM((STAGE_SUB,), jnp.float32),
        pltpu.VMEM((STAGE_SUB,), jnp.float32),
        pltpu.VMEM((CH,), jnp.float32),
        pltpu.VMEM_SHARED((VP,), jnp.float32),
        pltpu.SemaphoreType.DMA,
        pltpu.SemaphoreType.DMA,
        pltpu.SemaphoreType.DMA,
    ],
    compiler_params=pltpu.CompilerParams(needs_layout_passes=False),
)
def _sc_pool(idx_hbm, t2_hbm, out_hbm, idx_t, idx_v0, idx_v1, vals_v0,
             vals_v1, stage_v0, stage_v1, out_v, t2_sh, sem, sem2, sem3):
    cid = lax.axis_index("c")
    sid = lax.axis_index("s")
    wid = sid * NC + cid
    idx_vs = (idx_v0, idx_v1)
    vals_vs = (vals_v0, vals_v1)

    def fire_idx(c):
        col0 = wid * ROWS_W + c * CH
        return pltpu.async_copy(idx_hbm.at[:, pl.ds(col0, CH)], idx_t, sem2)

    idx_dma = fire_idx(0)

    for k in range(STAGE // STAGE_SUB):
        off = sid * STAGE + k * STAGE_SUB
        pltpu.sync_copy(t2_hbm.at[pl.ds(off, STAGE_SUB)], stage_v0)
        pltpu.sync_copy(stage_v0, t2_sh.at[pl.ds(off, STAGE_SUB)])

    plsc.subcore_barrier()

    NG = CH // 16
    for c in range(NCHUNK):
        col0 = wid * ROWS_W + c * CH
        idx_dma.wait()

        accs = [jnp.zeros((16,), jnp.float32)] * NG

        def reduce_pass(b, accs):
            vv = vals_vs[b]

            def body(s, a):
                return tuple(a[g] + vv[pl.ds(s * CH + g * 16, 16)]
                             for g in range(NG))

            return list(lax.fori_loop(0, SH, body, tuple(accs)))

        gat = None
        for p in range(NPASS):
            b = p & 1
            iv = idx_vs[b]

            def repack(s, carry, p=p, iv=iv):
                for g in range(NG):
                    iv[pl.ds(s * CH + g * 16, 16)] = (
                        idx_t[p * SH + s, pl.ds(g * 16, 16)])
                return carry

            lax.fori_loop(0, SH, repack, 0)
            d = pltpu.async_copy(t2_sh.at[iv], vals_vs[b], sem)
            if p == NPASS - 1 and c < NCHUNK - 1:
                idx_dma = fire_idx(c + 1)
            if gat is not None:
                gat.wait()
                accs = reduce_pass(1 - b, accs)
            gat = d
        gat.wait()
        accs = reduce_pass((NPASS - 1) & 1, accs)

        for g in range(NG):
            out_v[pl.ds(g * 16, 16)] = 1.0 / (1.0 + jnp.exp(-accs[g]))
        pltpu.sync_copy(out_v, out_hbm.at[pl.ds(col0, CH)])


def kernel(inputs, table, dense_w, dense_b):
    w_col = dense_w * (1.0 / S)
    bias = (dense_b * (1.0 / S)).reshape(1, 1)

    t2 = _fold_table(table.T, w_col, bias)

    out = _sc_pool(inputs.T, t2)
    return out.reshape(B, 1)
